# Initial kernel scaffold; baseline (speedup 1.0000x reference)
import jax
import jax.numpy as jnp
from jax.experimental import pallas as pl


def kernel(x, edge_index, node_type, edge_type, node_type_table, edge_type_table, WN_w, WN_b, WR_w, WR_b, A_w, A_b):
    raise NotImplementedError("write your pallas kernel here")


# speedup vs baseline: 5.7675x; 5.7675x over previous
import functools

import jax
import jax.numpy as jnp
from jax import lax
from jax.experimental import pallas as pl
from jax.experimental.pallas import tpu as pltpu
from jax.experimental.pallas import tpu_sc as plsc

NC = 2
NS = 16
NW = NC * NS
LANES = 16


def _sne_body(x_ref, nt_ref, tab_ref, out_ref):
    nt = nt_ref[0, 0, :]
    nrows = nt.shape[0]
    ntypes = tab_ref.shape[0]
    oh = (nt[:, None] == lax.broadcasted_iota(jnp.int32, (nrows, ntypes), 1))
    emb = jnp.dot(oh.astype(jnp.float32), tab_ref[...],
                  preferred_element_type=jnp.float32)
    out_ref[...] = x_ref[...] + emb


def _compute_sne(x, node_type, table):
    n, c = x.shape
    blk = 1000
    grid = n // blk
    nt3 = node_type.astype(jnp.int32).reshape(grid, 1, blk)
    return pl.pallas_call(
        _sne_body,
        grid=(grid,),
        in_specs=[
            pl.BlockSpec((blk, c), lambda i: (i, 0)),
            pl.BlockSpec((1, 1, blk), lambda i: (i, 0, 0)),
            pl.BlockSpec(table.shape, lambda i: (0, 0)),
        ],
        out_specs=pl.BlockSpec((blk, c), lambda i: (i, 0)),
        out_shape=jax.ShapeDtypeStruct((n, c), jnp.float32),
    )(x, nt3, table)


def _scatter_sc(sne, src_r, dst_r, et_r):
    n, c = sne.shape
    ns, nch, bch = src_r.shape
    half = n // NC
    hrows = half + 8
    stripe = half // NS // 8 * 8
    tail = half - NS * stripe
    zstripe = hrows // NS // 8 * 8
    ztail = hrows - NS * zstripe
    hsize = hrows * LANES
    hstripe = stripe * LANES
    htail = tail * LANES
    mesh = plsc.VectorSubcoreMesh(core_axis_name="c", subcore_axis_name="s",
                                  num_cores=NC, num_subcores=NS)

    @functools.partial(
        pl.kernel,
        out_type=[
            jax.ShapeDtypeStruct((n, c), jnp.float32),
            jax.ShapeDtypeStruct((n * LANES,), jnp.float32),
        ],
        mesh=mesh,
        scratch_types=[
            pltpu.VMEM((bch,), jnp.int32),
            pltpu.VMEM((bch,), jnp.int32),
            pltpu.VMEM((bch,), jnp.int32),
            pltpu.VMEM((bch, c), jnp.float32),
            pltpu.VMEM((bch,), jnp.float32),
            pltpu.VMEM((bch,), jnp.int32),
            pltpu.VMEM((bch,), jnp.int32),
            pltpu.VMEM((bch * LANES,), jnp.float32),
            pltpu.VMEM_SHARED((hrows, c), jnp.float32),
            pltpu.VMEM_SHARED((hsize,), jnp.float32),
            pltpu.SemaphoreType.DMA,
        ],
    )
    def k(sne_hbm, src_hbm, dst_hbm, et_hbm, s_out, h_out,
          src_v, dst_v, et_v, rows_v, ones_v, dmod_v, hidx_v, zflat_v,
          s_sh, h_sh, sem):
        cid = lax.axis_index("c")
        sid = lax.axis_index("s")
        lo = cid * half

        zero16 = jnp.zeros((LANES,), jnp.float32)
        one16 = jnp.ones((LANES,), jnp.float32)

        def zrow(i, carry):
            def zcol(j, carry2):
                rows_v[i, pl.ds(j * LANES, LANES)] = zero16
                return carry2
            lax.fori_loop(0, c // LANES, zcol, 0)
            return carry
        lax.fori_loop(0, bch, zrow, 0)

        def fill_ones(g, carry):
            ones_v[pl.ds(g * LANES, LANES)] = one16
            return carry
        lax.fori_loop(0, bch // LANES, fill_ones, 0)

        def fill_zeros(g, carry):
            zflat_v[pl.ds(g * LANES, LANES)] = zero16
            return carry
        lax.fori_loop(0, bch, fill_zeros, 0)

        off = 0
        while off < zstripe:
            sz = min(bch, zstripe - off)
            pltpu.sync_copy(rows_v.at[pl.ds(0, sz)],
                            s_sh.at[pl.ds(sid * zstripe + off, sz)])
            off += sz
        zh = zstripe * LANES
        off = 0
        while off < zh:
            sz = min(bch * LANES, zh - off)
            pltpu.sync_copy(zflat_v.at[pl.ds(0, sz)],
                            h_sh.at[pl.ds(sid * zh + off, sz)])
            off += sz
        if ztail:
            @pl.when(sid == NS - 1)
            def _ztail():
                zoff = NS * zstripe
                pltpu.sync_copy(rows_v.at[pl.ds(0, ztail)],
                                s_sh.at[pl.ds(zoff, ztail)])
                zt = ztail * LANES
                pltpu.sync_copy(zflat_v.at[pl.ds(0, zt)],
                                h_sh.at[pl.ds(zoff * LANES, zt)])
        plsc.subcore_barrier()

        ngroups = bch // LANES

        def chunk(kk, carry):
            pltpu.sync_copy(src_hbm.at[sid, kk], src_v)
            pltpu.sync_copy(dst_hbm.at[sid, kk], dst_v)
            pltpu.sync_copy(et_hbm.at[sid, kk], et_v)
            pltpu.async_copy(sne_hbm.at[src_v], rows_v, sem).wait()

            def clamp(g, carry2):
                d16 = dst_v[pl.ds(g * LANES, LANES)]
                e16 = et_v[pl.ds(g * LANES, LANES)]
                dloc = d16 - lo
                inr = (dloc >= 0) & (dloc < half)
                deff = jnp.where(inr, dloc, half)
                dmod_v[pl.ds(g * LANES, LANES)] = deff
                hidx_v[pl.ds(g * LANES, LANES)] = jnp.where(
                    inr, dloc * LANES + e16, half * LANES)
                return carry2
            lax.fori_loop(0, ngroups, clamp, 0)

            pltpu.sync_copy(rows_v, s_sh.at[dmod_v], add=True)
            pltpu.sync_copy(ones_v, h_sh.at[hidx_v], add=True)
            return carry
        lax.fori_loop(0, nch, chunk, 0)

        plsc.subcore_barrier()
        pltpu.sync_copy(s_sh.at[pl.ds(sid * stripe, stripe)],
                        s_out.at[pl.ds(lo + sid * stripe, stripe)])
        pltpu.sync_copy(h_sh.at[pl.ds(sid * hstripe, hstripe)],
                        h_out.at[pl.ds(lo * LANES + sid * hstripe, hstripe)])
        if tail:
            @pl.when(sid == NS - 1)
            def _wtail():
                pltpu.sync_copy(s_sh.at[pl.ds(NS * stripe, tail)],
                                s_out.at[pl.ds(lo + NS * stripe, tail)])
                pltpu.sync_copy(h_sh.at[pl.ds(NS * hstripe, htail)],
                                h_out.at[pl.ds(lo * LANES + NS * hstripe, htail)])

    return k(sne, src_r, dst_r, et_r)


def _combine_body(sne_ref, s2_ref, h2_ref, ete_ref, w1_ref, w2_ref, wnb_ref,
                  wrb_ref, a1_ref, a2_ref, ab_ref,
                  s_ref, t_ref, ge_ref, sm_ref, tm_ref, *, nblocks, n):
    i = pl.program_id(0)
    s_acc = s2_ref[...]
    hist = h2_ref[...]
    cnt = jnp.sum(hist, axis=1, keepdims=True)
    tsum = jnp.dot(hist, ete_ref[...], preferred_element_type=jnp.float32)
    denom = jnp.maximum(cnt, 1.0)
    efm = (s_acc - tsum) / denom
    h = (jnp.dot(sne_ref[...], w1_ref[...], preferred_element_type=jnp.float32)
         + jnp.dot(efm, w2_ref[...], preferred_element_type=jnp.float32)
         + wnb_ref[...]
         + jnp.where(cnt > 0.0, 1.0, 0.0) * wrb_ref[...])
    s = jnp.dot(h, a1_ref[...], preferred_element_type=jnp.float32)
    t = jnp.dot(h, a2_ref[...], preferred_element_type=jnp.float32) + ab_ref[0, 0]
    s_ref[...] = s
    t_ref[...] = t

    @pl.when(i == 0)
    def _init():
        ge_ref[...] = jnp.zeros_like(ge_ref)
        sm_ref[...] = jnp.full_like(sm_ref, -jnp.inf)
        tm_ref[...] = jnp.full_like(tm_ref, -jnp.inf)

    ge_ref[...] += jnp.sum(h, axis=0, keepdims=True)
    sm_ref[...] = jnp.maximum(sm_ref[...], jnp.max(s))
    tm_ref[...] = jnp.maximum(tm_ref[...], jnp.max(t))

    @pl.when(i == nblocks - 1)
    def _fin():
        ge_ref[...] = ge_ref[...] * (1.0 / n)


def _combine(sne, s2, h2, ete16, w1, w2, wnb, wrb, a1, a2, ab):
    n, c = sne.shape
    out = w1.shape[1]
    blk = 1000
    grid = n // blk
    return pl.pallas_call(
        functools.partial(_combine_body, nblocks=grid, n=n),
        grid=(grid,),
        in_specs=[
            pl.BlockSpec((blk, c), lambda i: (i, 0)),
            pl.BlockSpec((blk, c), lambda i: (i, 0)),
            pl.BlockSpec((blk, LANES), lambda i: (i, 0)),
            pl.BlockSpec(ete16.shape, lambda i: (0, 0)),
            pl.BlockSpec(w1.shape, lambda i: (0, 0)),
            pl.BlockSpec(w2.shape, lambda i: (0, 0)),
            pl.BlockSpec(wnb.shape, lambda i: (0, 0)),
            pl.BlockSpec(wrb.shape, lambda i: (0, 0)),
            pl.BlockSpec(a1.shape, lambda i: (0, 0)),
            pl.BlockSpec(a2.shape, lambda i: (0, 0)),
            pl.BlockSpec(ab.shape, lambda i: (0, 0)),
        ],
        out_specs=[
            pl.BlockSpec((blk, 1), lambda i: (i, 0)),
            pl.BlockSpec((blk, 1), lambda i: (i, 0)),
            pl.BlockSpec((1, out), lambda i: (0, 0)),
            pl.BlockSpec((1, 1), lambda i: (0, 0)),
            pl.BlockSpec((1, 1), lambda i: (0, 0)),
        ],
        out_shape=[
            jax.ShapeDtypeStruct((n, 1), jnp.float32),
            jax.ShapeDtypeStruct((n, 1), jnp.float32),
            jax.ShapeDtypeStruct((1, out), jnp.float32),
            jax.ShapeDtypeStruct((1, 1), jnp.float32),
            jax.ShapeDtypeStruct((1, 1), jnp.float32),
        ],
    )(sne, s2, h2, ete16, w1, w2, wnb, wrb, a1, a2, ab)


def _attn_sc(s_arr, t_arr, src_a, dst_a, shift):
    n = s_arr.shape[0]
    nw, per_w = src_a.shape
    mesh = plsc.VectorSubcoreMesh(core_axis_name="c", subcore_axis_name="s", num_cores=NC, num_subcores=NS)

    @functools.partial(
        pl.kernel,
        out_type=[
            jax.ShapeDtypeStruct((nw, per_w), jnp.float32),
            jax.ShapeDtypeStruct((nw, LANES), jnp.float32),
        ],
        mesh=mesh,
        scratch_types=[
            pltpu.VMEM((n,), jnp.float32),
            pltpu.VMEM((n,), jnp.float32),
            pltpu.VMEM((per_w,), jnp.int32),
            pltpu.VMEM((per_w,), jnp.int32),
            pltpu.VMEM((per_w,), jnp.float32),
            pltpu.VMEM((LANES,), jnp.float32),
        ],
        compiler_params=pltpu.CompilerParams(needs_layout_passes=False),
    )
    def k(s_hbm, t_hbm, src_hbm, dst_hbm, sh_hbm, w_out, ps_out,
          s_v, t_v, src_v, dst_v, w_v, acc_v):
        cid = lax.axis_index("c")
        sid = lax.axis_index("s")
        wid = sid * NC + cid
        pltpu.sync_copy(s_hbm, s_v)
        pltpu.sync_copy(t_hbm, t_v)
        pltpu.sync_copy(sh_hbm, acc_v)
        pltpu.sync_copy(src_hbm.at[wid], src_v)
        pltpu.sync_copy(dst_hbm.at[wid], dst_v)
        sh = acc_v[...]

        def body(g, acc):
            si = src_v[pl.ds(g * LANES, LANES)]
            di = dst_v[pl.ds(g * LANES, LANES)]
            sv = plsc.load_gather(s_v, [si])
            tv = plsc.load_gather(t_v, [di])
            wv = jnp.exp(sv + tv - sh)
            w_v[pl.ds(g * LANES, LANES)] = wv
            return acc + wv
        acc = lax.fori_loop(0, per_w // LANES, body,
                            jnp.zeros((LANES,), jnp.float32))
        acc_v[...] = acc
        pltpu.sync_copy(w_v, w_out.at[wid])
        pltpu.sync_copy(acc_v, ps_out.at[wid])

    return k(s_arr, t_arr, src_a, dst_a, shift)


def _norm_body(w_ref, ps_ref, out_ref):
    z = jnp.sum(ps_ref[...])
    out_ref[...] = w_ref[...] * (1.0 / z)


def _normalize(w2d, psums):
    return pl.pallas_call(
        _norm_body,
        out_shape=jax.ShapeDtypeStruct(w2d.shape, jnp.float32),
    )(w2d, psums)


def kernel(x, edge_index, node_type, edge_type, node_type_table,
           edge_type_table, WN_w, WN_b, WR_w, WR_b, A_w, A_b):
    n, c = x.shape
    e = edge_index.shape[1]
    out = WN_w.shape[0]

    sne = _compute_sne(x, node_type, node_type_table)

    bch = 80
    per_t = e // NS
    nch = per_t // bch
    src = edge_index[0].astype(jnp.int32)
    dst = edge_index[1].astype(jnp.int32)
    src_r = src.reshape(NS, nch, bch)
    dst_r = dst.reshape(NS, nch, bch)
    et_r = edge_type.astype(jnp.int32).reshape(NS, nch, bch)

    s2, h2f = _scatter_sc(sne, src_r, dst_r, et_r)
    h2 = h2f.reshape(n, LANES)

    net = edge_type_table.shape[0]
    ete16 = jnp.zeros((LANES, c), jnp.float32).at[:net].set(edge_type_table)
    w1 = WN_w[:, :c].T
    w2 = (WN_w[:, c:] + WR_w).T
    a1 = A_w[0, :out].reshape(out, 1)
    a2 = A_w[0, out:].reshape(out, 1)
    ab = A_b.reshape(1, 1)
    wnb = WN_b.reshape(1, out)
    wrb = WR_b.reshape(1, out)

    s_col, t_col, ge, sm, tm = _combine(
        sne, s2, h2, ete16, w1, w2, wnb, wrb, a1, a2, ab)

    shift = jnp.full((LANES,), sm[0, 0] + tm[0, 0], jnp.float32)
    per_w = e // NW
    w_un, psums = _attn_sc(s_col.reshape(n), t_col.reshape(n),
                           src.reshape(NW, per_w), dst.reshape(NW, per_w),
                           shift)

    att = _normalize(w_un.reshape(e // 128, 128), psums).reshape(e)
    return (ge.reshape(out), att)


# speedup vs baseline: 11.8226x; 2.0499x over previous
import functools

import jax
import jax.numpy as jnp
from jax import lax
from jax.experimental import pallas as pl
from jax.experimental.pallas import tpu as pltpu
from jax.experimental.pallas import tpu_sc as plsc

NC = 2
NS = 16
NW = NC * NS
LANES = 16


def _sne_body(x_ref, nt_ref, tab_ref, out_ref):
    nt = nt_ref[0, 0, :]
    nrows = nt.shape[0]
    ntypes = tab_ref.shape[0]
    oh = (nt[:, None] == lax.broadcasted_iota(jnp.int32, (nrows, ntypes), 1))
    emb = jnp.dot(oh.astype(jnp.float32), tab_ref[...],
                  preferred_element_type=jnp.float32)
    out_ref[...] = x_ref[...] + emb


def _compute_sne(x, node_type, table):
    n, c = x.shape
    blk = 1000
    grid = n // blk
    nt3 = node_type.astype(jnp.int32).reshape(grid, 1, blk)
    return pl.pallas_call(
        _sne_body,
        grid=(grid,),
        in_specs=[
            pl.BlockSpec((blk, c), lambda i: (i, 0)),
            pl.BlockSpec((1, 1, blk), lambda i: (i, 0, 0)),
            pl.BlockSpec(table.shape, lambda i: (0, 0)),
        ],
        out_specs=pl.BlockSpec((blk, c), lambda i: (i, 0)),
        out_shape=jax.ShapeDtypeStruct((n, c), jnp.float32),
    )(x, nt3, table)


def _scatter_sc(sne, idx3):
    n, c = sne.shape
    ns, nch, bch3 = idx3.shape
    bch = bch3 // 3
    half = n // NC
    hrows = half + 8
    stripe = half // NS // 8 * 8
    tail = half - NS * stripe
    zstripe = hrows // NS // 8 * 8
    ztail = hrows - NS * zstripe
    hsize = hrows * LANES
    hstripe = stripe * LANES
    htail = tail * LANES
    mesh = plsc.VectorSubcoreMesh(core_axis_name="c", subcore_axis_name="s",
                                  num_cores=NC, num_subcores=NS)

    @functools.partial(
        pl.kernel,
        out_type=[
            jax.ShapeDtypeStruct((n, c), jnp.float32),
            jax.ShapeDtypeStruct((n * LANES,), jnp.float32),
        ],
        mesh=mesh,
        scratch_types=[
            pltpu.VMEM((3 * bch,), jnp.int32),
            pltpu.VMEM((3 * bch,), jnp.int32),
            pltpu.VMEM((bch, c), jnp.float32),
            pltpu.VMEM((bch, c), jnp.float32),
            pltpu.VMEM((bch,), jnp.float32),
            pltpu.VMEM((bch,), jnp.int32),
            pltpu.VMEM((bch,), jnp.int32),
            pltpu.VMEM((bch * LANES,), jnp.float32),
            pltpu.VMEM_SHARED((hrows, c), jnp.float32),
            pltpu.VMEM_SHARED((hsize,), jnp.float32),
            pltpu.SemaphoreType.DMA,
            pltpu.SemaphoreType.DMA,
            pltpu.SemaphoreType.DMA,
            pltpu.SemaphoreType.DMA,
        ],
    )
    def k(sne_hbm, idx3_hbm, s_out, h_out,
          idx_v0, idx_v1, rows_v0, rows_v1, ones_v, dmod_v, hidx_v, zflat_v,
          s_sh, h_sh, semi0, semi1, semg0, semg1):
        idx_v = (idx_v0, idx_v1)
        rows_v = (rows_v0, rows_v1)
        semi = (semi0, semi1)
        semg = (semg0, semg1)
        cid = lax.axis_index("c")
        sid = lax.axis_index("s")
        lo = cid * half

        zero16 = jnp.zeros((LANES,), jnp.float32)
        one16 = jnp.ones((LANES,), jnp.float32)

        def zrow(i, carry):
            def zcol(j, carry2):
                rows_v0[i, pl.ds(j * LANES, LANES)] = zero16
                return carry2
            lax.fori_loop(0, c // LANES, zcol, 0)
            return carry
        lax.fori_loop(0, bch, zrow, 0)

        def fill_ones(g, carry):
            ones_v[pl.ds(g * LANES, LANES)] = one16
            return carry
        lax.fori_loop(0, bch // LANES, fill_ones, 0)

        def fill_zeros(g, carry):
            zflat_v[pl.ds(g * LANES, LANES)] = zero16
            return carry
        lax.fori_loop(0, bch, fill_zeros, 0)

        off = 0
        while off < zstripe:
            sz = min(bch, zstripe - off)
            pltpu.sync_copy(rows_v0.at[pl.ds(0, sz)],
                            s_sh.at[pl.ds(sid * zstripe + off, sz)])
            off += sz
        zh = zstripe * LANES
        off = 0
        while off < zh:
            sz = min(bch * LANES, zh - off)
            pltpu.sync_copy(zflat_v.at[pl.ds(0, sz)],
                            h_sh.at[pl.ds(sid * zh + off, sz)])
            off += sz
        if ztail:
            @pl.when(sid == NS - 1)
            def _ztail():
                zoff = NS * zstripe
                pltpu.sync_copy(rows_v0.at[pl.ds(0, ztail)],
                                s_sh.at[pl.ds(zoff, ztail)])
                zt = ztail * LANES
                pltpu.sync_copy(zflat_v.at[pl.ds(0, zt)],
                                h_sh.at[pl.ds(zoff * LANES, zt)])
        plsc.subcore_barrier()

        ngroups = bch // LANES

        pltpu.async_copy(idx3_hbm.at[sid, 0], idx_v[0], semi[0]).wait()
        pltpu.async_copy(sne_hbm.at[idx_v[0].at[pl.ds(0, bch)]],
                         rows_v[0], semg[0])
        pltpu.async_copy(idx3_hbm.at[sid, 1], idx_v[1], semi[1])

        def half_step(kk, b):
            pltpu.make_async_copy(sne_hbm.at[pl.ds(0, bch)],
                                  rows_v[b], semg[b]).wait()

            def clamp(g, carry2):
                d16 = idx_v[b][pl.ds(bch + g * LANES, LANES)]
                e16 = idx_v[b][pl.ds(2 * bch + g * LANES, LANES)]
                dloc = d16 - lo
                inr = (dloc >= 0) & (dloc < half)
                dmod_v[pl.ds(g * LANES, LANES)] = jnp.where(inr, dloc, half)
                hidx_v[pl.ds(g * LANES, LANES)] = jnp.where(
                    inr, dloc * LANES + e16, half * LANES)
                return carry2
            lax.fori_loop(0, ngroups, clamp, 0)

            @pl.when(kk + 1 < nch)
            def _next_gather():
                pltpu.make_async_copy(idx3_hbm.at[sid, 0],
                                      idx_v[1 - b], semi[1 - b]).wait()
                pltpu.async_copy(sne_hbm.at[idx_v[1 - b].at[pl.ds(0, bch)]],
                                 rows_v[1 - b], semg[1 - b])

            pltpu.sync_copy(rows_v[b], s_sh.at[dmod_v], add=True)
            pltpu.sync_copy(ones_v, h_sh.at[hidx_v], add=True)

            @pl.when(kk + 2 < nch)
            def _prefetch_idx():
                pltpu.async_copy(idx3_hbm.at[sid, kk + 2], idx_v[b], semi[b])

        def chunk_pair(i, carry):
            half_step(2 * i, 0)
            half_step(2 * i + 1, 1)
            return carry
        lax.fori_loop(0, nch // 2, chunk_pair, 0)

        plsc.subcore_barrier()
        pltpu.sync_copy(s_sh.at[pl.ds(sid * stripe, stripe)],
                        s_out.at[pl.ds(lo + sid * stripe, stripe)])
        pltpu.sync_copy(h_sh.at[pl.ds(sid * hstripe, hstripe)],
                        h_out.at[pl.ds(lo * LANES + sid * hstripe, hstripe)])
        if tail:
            @pl.when(sid == NS - 1)
            def _wtail():
                pltpu.sync_copy(s_sh.at[pl.ds(NS * stripe, tail)],
                                s_out.at[pl.ds(lo + NS * stripe, tail)])
                pltpu.sync_copy(h_sh.at[pl.ds(NS * hstripe, htail)],
                                h_out.at[pl.ds(lo * LANES + NS * hstripe, htail)])

    return k(sne, idx3)


def _combine_body(sne_ref, s2_ref, h2_ref, ete_ref, w1_ref, w2_ref, wnb_ref,
                  wrb_ref, a1_ref, a2_ref, ab_ref,
                  s_ref, t_ref, ge_ref, sm_ref, tm_ref, *, nblocks, n):
    i = pl.program_id(0)
    s_acc = s2_ref[...]
    hist = h2_ref[...]
    cnt = jnp.sum(hist, axis=1, keepdims=True)
    tsum = jnp.dot(hist, ete_ref[...], preferred_element_type=jnp.float32)
    denom = jnp.maximum(cnt, 1.0)
    efm = (s_acc - tsum) / denom
    h = (jnp.dot(sne_ref[...], w1_ref[...], preferred_element_type=jnp.float32)
         + jnp.dot(efm, w2_ref[...], preferred_element_type=jnp.float32)
         + wnb_ref[...]
         + jnp.where(cnt > 0.0, 1.0, 0.0) * wrb_ref[...])
    s = jnp.dot(h, a1_ref[...], preferred_element_type=jnp.float32)
    t = jnp.dot(h, a2_ref[...], preferred_element_type=jnp.float32) + ab_ref[0, 0]
    s_ref[...] = s
    t_ref[...] = t

    @pl.when(i == 0)
    def _init():
        ge_ref[...] = jnp.zeros_like(ge_ref)
        sm_ref[...] = jnp.full_like(sm_ref, -jnp.inf)
        tm_ref[...] = jnp.full_like(tm_ref, -jnp.inf)

    ge_ref[...] += jnp.sum(h, axis=0, keepdims=True)
    sm_ref[...] = jnp.maximum(sm_ref[...], jnp.max(s))
    tm_ref[...] = jnp.maximum(tm_ref[...], jnp.max(t))

    @pl.when(i == nblocks - 1)
    def _fin():
        ge_ref[...] = ge_ref[...] * (1.0 / n)


def _combine(sne, s2, h2, ete16, w1, w2, wnb, wrb, a1, a2, ab):
    n, c = sne.shape
    out = w1.shape[1]
    blk = 1000
    grid = n // blk
    return pl.pallas_call(
        functools.partial(_combine_body, nblocks=grid, n=n),
        grid=(grid,),
        in_specs=[
            pl.BlockSpec((blk, c), lambda i: (i, 0)),
            pl.BlockSpec((blk, c), lambda i: (i, 0)),
            pl.BlockSpec((blk, LANES), lambda i: (i, 0)),
            pl.BlockSpec(ete16.shape, lambda i: (0, 0)),
            pl.BlockSpec(w1.shape, lambda i: (0, 0)),
            pl.BlockSpec(w2.shape, lambda i: (0, 0)),
            pl.BlockSpec(wnb.shape, lambda i: (0, 0)),
            pl.BlockSpec(wrb.shape, lambda i: (0, 0)),
            pl.BlockSpec(a1.shape, lambda i: (0, 0)),
            pl.BlockSpec(a2.shape, lambda i: (0, 0)),
            pl.BlockSpec(ab.shape, lambda i: (0, 0)),
        ],
        out_specs=[
            pl.BlockSpec((blk, 1), lambda i: (i, 0)),
            pl.BlockSpec((blk, 1), lambda i: (i, 0)),
            pl.BlockSpec((1, out), lambda i: (0, 0)),
            pl.BlockSpec((1, 1), lambda i: (0, 0)),
            pl.BlockSpec((1, 1), lambda i: (0, 0)),
        ],
        out_shape=[
            jax.ShapeDtypeStruct((n, 1), jnp.float32),
            jax.ShapeDtypeStruct((n, 1), jnp.float32),
            jax.ShapeDtypeStruct((1, out), jnp.float32),
            jax.ShapeDtypeStruct((1, 1), jnp.float32),
            jax.ShapeDtypeStruct((1, 1), jnp.float32),
        ],
    )(sne, s2, h2, ete16, w1, w2, wnb, wrb, a1, a2, ab)


def _attn_sc(s_arr, t_arr, src_a, dst_a, shift):
    n = s_arr.shape[0]
    nw, per_w = src_a.shape
    mesh = plsc.VectorSubcoreMesh(core_axis_name="c", subcore_axis_name="s", num_cores=NC, num_subcores=NS)

    @functools.partial(
        pl.kernel,
        out_type=[
            jax.ShapeDtypeStruct((nw, per_w), jnp.float32),
            jax.ShapeDtypeStruct((nw, LANES), jnp.float32),
        ],
        mesh=mesh,
        scratch_types=[
            pltpu.VMEM((n,), jnp.float32),
            pltpu.VMEM((n,), jnp.float32),
            pltpu.VMEM((per_w,), jnp.int32),
            pltpu.VMEM((per_w,), jnp.int32),
            pltpu.VMEM((per_w,), jnp.float32),
            pltpu.VMEM((LANES,), jnp.float32),
        ],
        compiler_params=pltpu.CompilerParams(needs_layout_passes=False),
    )
    def k(s_hbm, t_hbm, src_hbm, dst_hbm, sh_hbm, w_out, ps_out,
          s_v, t_v, src_v, dst_v, w_v, acc_v):
        cid = lax.axis_index("c")
        sid = lax.axis_index("s")
        wid = sid * NC + cid
        pltpu.sync_copy(s_hbm, s_v)
        pltpu.sync_copy(t_hbm, t_v)
        pltpu.sync_copy(sh_hbm, acc_v)
        pltpu.sync_copy(src_hbm.at[wid], src_v)
        pltpu.sync_copy(dst_hbm.at[wid], dst_v)
        sh = acc_v[...]

        def body(g, acc):
            si = src_v[pl.ds(g * LANES, LANES)]
            di = dst_v[pl.ds(g * LANES, LANES)]
            sv = plsc.load_gather(s_v, [si])
            tv = plsc.load_gather(t_v, [di])
            wv = jnp.exp(sv + tv - sh)
            w_v[pl.ds(g * LANES, LANES)] = wv
            return acc + wv
        acc = lax.fori_loop(0, per_w // LANES, body,
                            jnp.zeros((LANES,), jnp.float32))
        acc_v[...] = acc
        pltpu.sync_copy(w_v, w_out.at[wid])
        pltpu.sync_copy(acc_v, ps_out.at[wid])

    return k(s_arr, t_arr, src_a, dst_a, shift)


def _norm_body(w_ref, ps_ref, out_ref):
    z = jnp.sum(ps_ref[...])
    out_ref[...] = w_ref[...] * (1.0 / z)


def _normalize(w2d, psums):
    return pl.pallas_call(
        _norm_body,
        out_shape=jax.ShapeDtypeStruct(w2d.shape, jnp.float32),
    )(w2d, psums)


def kernel(x, edge_index, node_type, edge_type, node_type_table,
           edge_type_table, WN_w, WN_b, WR_w, WR_b, A_w, A_b):
    n, c = x.shape
    e = edge_index.shape[1]
    out = WN_w.shape[0]

    sne = _compute_sne(x, node_type, node_type_table)

    bch = 80
    per_t = e // NS
    nch = per_t // bch
    src = edge_index[0].astype(jnp.int32)
    dst = edge_index[1].astype(jnp.int32)
    idx3 = jnp.concatenate(
        [src.reshape(NS, nch, bch), dst.reshape(NS, nch, bch),
         edge_type.astype(jnp.int32).reshape(NS, nch, bch)], axis=-1)

    s2, h2f = _scatter_sc(sne, idx3)
    h2 = h2f.reshape(n, LANES)

    net = edge_type_table.shape[0]
    ete16 = jnp.zeros((LANES, c), jnp.float32).at[:net].set(edge_type_table)
    w1 = WN_w[:, :c].T
    w2 = (WN_w[:, c:] + WR_w).T
    a1 = A_w[0, :out].reshape(out, 1)
    a2 = A_w[0, out:].reshape(out, 1)
    ab = A_b.reshape(1, 1)
    wnb = WN_b.reshape(1, out)
    wrb = WR_b.reshape(1, out)

    s_col, t_col, ge, sm, tm = _combine(
        sne, s2, h2, ete16, w1, w2, wnb, wrb, a1, a2, ab)

    shift = jnp.full((LANES,), sm[0, 0] + tm[0, 0], jnp.float32)
    per_w = e // NW
    w_un, psums = _attn_sc(s_col.reshape(n), t_col.reshape(n),
                           src.reshape(NW, per_w), dst.reshape(NW, per_w),
                           shift)

    att = _normalize(w_un.reshape(e // 128, 128), psums).reshape(e)
    return (ge.reshape(out), att)
